# baseline (speedup 1.0000x reference)
import jax
import jax.numpy as jnp
from jax.experimental import pallas as pl
from jax.experimental.pallas import tpu as pltpu

N = 8192
BR = 2048
BC = 2048


def _bt_block(r_rows_ref, e_cols_ref, out_ref):
    den = e_cols_ref[...] * r_rows_ref[...] + 1.0
    out_ref[...] = 1.0 / den


def kernel(ability):
    r_rows = jnp.exp(-ability).reshape(N, 1)
    e_cols = jnp.exp(ability).reshape(1, N)
    return pl.pallas_call(
        _bt_block,
        grid=(N // BR, N // BC),
        in_specs=[
            pl.BlockSpec((BR, 1), lambda i, j: (i, 0)),
            pl.BlockSpec((1, BC), lambda i, j: (0, j)),
        ],
        out_specs=pl.BlockSpec((BR, BC), lambda i, j: (i, j)),
        out_shape=jax.ShapeDtypeStruct((N, N), jnp.float32),
    )(r_rows, e_cols)


# speedup vs baseline: 1.0167x; 1.0167x over previous
import jax
import jax.numpy as jnp
from jax.experimental import pallas as pl
from jax.experimental.pallas import tpu as pltpu

N = 8192
BR = 512


def _bt_block(r_rows_ref, e_cols_ref, out_ref):
    den = e_cols_ref[...] * r_rows_ref[...] + 1.0
    out_ref[...] = 1.0 / den


def kernel(ability):
    r_rows = jnp.exp(-ability).reshape(N, 1)
    e_cols = jnp.exp(ability).reshape(1, N)
    return pl.pallas_call(
        _bt_block,
        grid=(N // BR,),
        in_specs=[
            pl.BlockSpec((BR, 1), lambda i: (i, 0)),
            pl.BlockSpec((1, N), lambda i: (0, 0)),
        ],
        out_specs=pl.BlockSpec((BR, N), lambda i: (i, 0)),
        out_shape=jax.ShapeDtypeStruct((N, N), jnp.float32),
        compiler_params=pltpu.CompilerParams(skip_device_barrier=True),
    )(r_rows, e_cols)
